# baseline (speedup 1.0000x reference)
import functools

import jax
import jax.numpy as jnp
from jax import lax
from jax.experimental import pallas as pl
from jax.experimental.pallas import tpu as pltpu
from jax.experimental.pallas import tpu_sc as plsc

N = 10000
NP = 10112
H = 128
E = 640000
NSUB = 16
CHUNK = 64
KST = 8
QETOT = KST * CHUNK
NQ = 1280
EPAD = NQ * QETOT
ROWS_PER_SUB = NP // NSUB
NWQ = NQ // 32
NCH = NWQ * KST
QDEG = NWQ

_mesh = plsc.VectorSubcoreMesh(core_axis_name="c", subcore_axis_name="s")


@functools.partial(
    pl.kernel,
    out_type=jax.ShapeDtypeStruct((2, NP, H), jnp.float32),
    mesh=_mesh,
    scratch_types=[
        pltpu.VMEM((3, KST, CHUNK), jnp.int32),
        pltpu.VMEM((3, KST, CHUNK), jnp.int32),
        pltpu.VMEM((4, CHUNK, H), jnp.float32),
        pltpu.VMEM_SHARED((NP, H), jnp.float32),
        pltpu.SemaphoreType.DMA,
        pltpu.SemaphoreType.DMA,
        pltpu.SemaphoreType.DMA,
        pltpu.SemaphoreType.DMA,
        pltpu.SemaphoreType.DMA,
        pltpu.SemaphoreType.DMA,
        pltpu.SemaphoreType.DMA,
        pltpu.SemaphoreType.DMA,
        pltpu.SemaphoreType.DMA,
    ],
)
def _segsum(ht, srcs, dsts, zeros, out, sidx, didx, gbuf, acc,
            sg0, sg1, sg2, sg3, ss0, ss1, ss2, ss3, si):
  SG = (sg0, sg1, sg2, sg3)
  SS = (ss0, ss1, ss2, ss3)
  cid = lax.axis_index("c")
  sid = lax.axis_index("s")
  wid = sid * 2 + cid
  q0 = wid * NWQ
  r0 = sid * ROWS_PER_SUB
  pltpu.sync_copy(zeros.at[pl.ds(r0, ROWS_PER_SUB)],
                  acc.at[pl.ds(r0, ROWS_PER_SUB)])
  pltpu.sync_copy(srcs.at[q0], sidx.at[0])
  pltpu.sync_copy(dsts.at[q0], didx.at[0])
  pltpu.async_copy(srcs.at[q0 + 1], sidx.at[1], si)
  pltpu.async_copy(dsts.at[q0 + 1], didx.at[1], si)
  plsc.subcore_barrier()
  for c in range(3):
    pltpu.async_copy(ht.at[sidx.at[0, c]], gbuf.at[c], SG[c])

  def quantum(k, carry):
    slot = k % 3
    nslot = (k + 1) % 3

    @pl.when(k + 1 < NWQ)
    def _idx_arrive():
      pltpu.make_async_copy(srcs.at[q0], sidx.at[nslot], si).wait()
      pltpu.make_async_copy(dsts.at[q0], didx.at[nslot], si).wait()

    for r in range(KST):
      c = k * KST + r
      b = r % 4
      pltpu.make_async_copy(ht.at[sidx.at[slot, r]], gbuf.at[b],
                            SG[b]).wait()
      pltpu.async_copy(gbuf.at[b], acc.at[didx.at[slot, r]], SS[b],
                       add=True)

      @pl.when(c + 3 < NCH)
      def _issue():
        b3 = (r + 3) % 4

        @pl.when(c > 0)
        def _reuse():
          pltpu.make_async_copy(gbuf.at[b3], acc.at[didx.at[slot, r]],
                                SS[b3]).wait()

        if r < KST - 3:
          pltpu.async_copy(ht.at[sidx.at[slot, r + 3]], gbuf.at[b3],
                           SG[b3])
        else:
          pltpu.async_copy(ht.at[sidx.at[nslot, r + 3 - KST]], gbuf.at[b3],
                           SG[b3])

    @pl.when(k + 2 < NWQ)
    def _prefetch():
      pslot = (k + 2) % 3
      pltpu.async_copy(srcs.at[q0 + k + 2], sidx.at[pslot], si)
      pltpu.async_copy(dsts.at[q0 + k + 2], didx.at[pslot], si)

    return carry

  lax.fori_loop(0, NWQ, quantum, 0)
  for b in range(4):
    pltpu.make_async_copy(gbuf.at[b], acc.at[didx.at[0, 0]], SS[b]).wait()
  plsc.subcore_barrier()
  pltpu.sync_copy(acc.at[pl.ds(r0, ROWS_PER_SUB)],
                  out.at[cid, pl.ds(r0, ROWS_PER_SUB)])


@functools.partial(
    pl.kernel,
    out_type=jax.ShapeDtypeStruct((2, NP, 16), jnp.float32),
    mesh=_mesh,
    scratch_types=[
        pltpu.VMEM((2, KST, CHUNK), jnp.int32),
        pltpu.VMEM((CHUNK, 16), jnp.float32),
        pltpu.VMEM_SHARED((NP, 16), jnp.float32),
        pltpu.SemaphoreType.DMA,
        pltpu.SemaphoreType.DMA,
    ],
)
def _deg_kernel(srcs, ones, zeros, out, sidx, obuf, acc, ss, si):
  cid = lax.axis_index("c")
  sid = lax.axis_index("s")
  q0 = (sid * 2 + cid) * QDEG
  r0 = sid * ROWS_PER_SUB
  pltpu.sync_copy(zeros.at[pl.ds(r0, ROWS_PER_SUB)],
                  acc.at[pl.ds(r0, ROWS_PER_SUB)])
  pltpu.sync_copy(ones, obuf)
  pltpu.sync_copy(srcs.at[q0], sidx.at[0])
  pltpu.async_copy(srcs.at[q0 + 1], sidx.at[1], si)
  plsc.subcore_barrier()

  def block(k, carry):
    slot = k % 2
    for r in range(KST):
      pltpu.async_copy(obuf, acc.at[sidx.at[slot, r]], ss, add=True)
    for r in range(KST):
      pltpu.make_async_copy(obuf, acc.at[sidx.at[slot, r]], ss).wait()

    @pl.when(k + 1 < QDEG)
    def _boundary():
      pltpu.make_async_copy(srcs.at[q0], sidx.at[1 - slot], si).wait()

      @pl.when(k + 2 < QDEG)
      def _prefetch():
        pltpu.async_copy(srcs.at[q0 + k + 2], sidx.at[slot], si)

    return carry

  lax.fori_loop(0, QDEG, block, 0)
  plsc.subcore_barrier()
  pltpu.sync_copy(acc.at[pl.ds(r0, ROWS_PER_SUB)],
                  out.at[cid, pl.ds(r0, ROWS_PER_SUB)])


_dot = functools.partial(lax.dot_general,
                         dimension_numbers=(((1,), (0,)), ((), ())),
                         preferred_element_type=jnp.float32)


def _prep_body(degp_ref, x16_ref, w1_ref, dis_ref, y2t_ref, y1_ref, y0c_ref):
  deg = degp_ref[0, :, 0:1] + degp_ref[1, :, 0:1]
  rowmask = lax.broadcasted_iota(jnp.int32, (NP, 1), 0) < N
  valid = jnp.logical_and(deg > 0.0, rowmask)
  dis = jnp.where(valid, lax.rsqrt(jnp.where(deg > 0.0, deg, 1.0)), 0.0)
  dis128 = jnp.broadcast_to(dis, (NP, H))
  dis_ref[...] = dis128
  x16 = x16_ref[...]
  y0 = _dot(x16, w1_ref[0])
  y1 = _dot(x16, w1_ref[1])
  y2 = _dot(x16, w1_ref[2])
  y2t_ref[...] = dis128 * y2
  y1_ref[...] = y1
  y0c_ref[...] = y0 - y2


_prep = pl.pallas_call(
    _prep_body,
    out_shape=[jax.ShapeDtypeStruct((NP, H), jnp.float32)] * 4,
)


def _mid1_body(sp_ref, dis_ref, y1_ref, zt_ref):
  d = dis_ref[...]
  zt_ref[...] = d * y1_ref[...] - 2.0 * d * d * (sp_ref[0] + sp_ref[1])


_mid1 = pl.pallas_call(
    _mid1_body,
    out_shape=jax.ShapeDtypeStruct((NP, H), jnp.float32),
)


def _bn_act(h, g, be, act):
  if act == "lrelu":
    h = jnp.where(h >= 0.0, h, 0.01 * h)
  else:
    h = jnp.maximum(h, 0.0)
  hv = h[0:N, :]
  mean = jnp.mean(hv, axis=0, keepdims=True)
  var = jnp.mean((hv - mean) ** 2, axis=0, keepdims=True)
  return (h - mean) * lax.rsqrt(var + 1e-5) * g + be


def _make_post1(act):
  def body(sp_ref, dis_ref, y0c_ref, b_ref, g_ref, be_ref, h_ref, ht_ref):
    dis = dis_ref[...]
    h = y0c_ref[...] - dis * (sp_ref[0] + sp_ref[1]) + b_ref[...]
    hbn = _bn_act(h, g_ref[...], be_ref[...], act)
    h_ref[...] = hbn
    ht_ref[...] = dis * hbn

  return pl.pallas_call(
      body,
      out_shape=[jax.ShapeDtypeStruct((NP, H), jnp.float32)] * 2,
  )


def _mid_body(sp_ref, dis_ref, tx1_ref, ht_ref):
  dis = dis_ref[...]
  tx1 = -dis * (sp_ref[0] + sp_ref[1])
  tx1_ref[...] = tx1
  ht_ref[...] = dis * tx1


_mid = pl.pallas_call(
    _mid_body,
    out_shape=[jax.ShapeDtypeStruct((NP, H), jnp.float32)] * 2,
)


def _cheb_out(sp_ref, dis_ref, tx0_ref, tx1_ref, w_ref, b_ref):
  tx0 = tx0_ref[...]
  tx2 = -2.0 * dis_ref[...] * (sp_ref[0] + sp_ref[1]) - tx0
  h = _dot(tx0, w_ref[0]) + _dot(tx1_ref[...], w_ref[1]) + _dot(tx2, w_ref[2])
  return h + b_ref[...]


def _make_post(act):
  def body(sp_ref, dis_ref, tx0_ref, tx1_ref, w_ref, b_ref, g_ref, be_ref,
           h_ref, ht_ref):
    h = _cheb_out(sp_ref, dis_ref, tx0_ref, tx1_ref, w_ref, b_ref)
    hbn = _bn_act(h, g_ref[...], be_ref[...], act)
    h_ref[...] = hbn
    ht_ref[...] = dis_ref[...] * hbn

  return pl.pallas_call(
      body,
      out_shape=[jax.ShapeDtypeStruct((NP, H), jnp.float32)] * 2,
  )


def _final_body(sp_ref, dis_ref, tx0_ref, tx1_ref, w_ref, b_ref, wr_ref,
                br_ref, out_ref):
  h = _cheb_out(sp_ref, dis_ref, tx0_ref, tx1_ref, w_ref, b_ref)
  h = h[0:N, :]
  nrm = jnp.sqrt(jnp.sum(h * h, axis=1, keepdims=True))
  h = h / jnp.maximum(nrm, 1e-12)
  out_ref[...] = _dot(h, wr_ref[...]) + br_ref[...]


_final = pl.pallas_call(
    _final_body,
    out_shape=jax.ShapeDtypeStruct((N, 3), jnp.float32),
)

_post1_lrelu = _make_post1("lrelu")
_post_lrelu = _make_post("lrelu")
_post_relu = _make_post("relu")


def kernel(x, edge_index, W1, b1, W2, b2, W3, b3, W4, b4,
           g1, be1, g2, be2, g3, be3, Wr, br):
  pad_e = EPAD - E
  src = jnp.concatenate([edge_index[0], jnp.full((pad_e,), N, jnp.int32)])
  dst = jnp.concatenate([edge_index[1], jnp.full((pad_e,), N, jnp.int32)])
  srcs = src.reshape(NQ, KST, CHUNK)
  dsts = dst.reshape(NQ, KST, CHUNK)

  x16 = jnp.pad(x, ((0, NP - N), (0, 13)))
  W1p = jnp.pad(W1, ((0, 0), (0, 13), (0, 0)))
  ones = jnp.ones((CHUNK, 16), jnp.float32)
  zeros = jnp.zeros((NP, H), jnp.float32)
  zeros16 = jnp.zeros((NP, 16), jnp.float32)
  b1r, b2r, b3r, b4r = (v.reshape(1, H) for v in (b1, b2, b3, b4))
  g1r, g2r, g3r = (v.reshape(1, H) for v in (g1, g2, g3))
  be1r, be2r, be3r = (v.reshape(1, H) for v in (be1, be2, be3))
  brr = br.reshape(1, 3)

  degp = _deg_kernel(srcs, ones, zeros16)
  dis, y2t, y1, y0c = _prep(degp, x16, W1p)

  sp = _segsum(y2t, srcs, dsts, zeros)
  zt = _mid1(sp, dis, y1)
  sp = _segsum(zt, srcs, dsts, zeros)
  h, ht = _post1_lrelu(sp, dis, y0c, b1r, g1r, be1r)

  for (W, br_, gr_, ber_, post) in (
      (W2, b2r, g2r, be2r, _post_lrelu),
      (W3, b3r, g3r, be3r, _post_relu),
  ):
    sp = _segsum(ht, srcs, dsts, zeros)
    tx1, htt = _mid(sp, dis)
    sp = _segsum(htt, srcs, dsts, zeros)
    h, ht = post(sp, dis, h, tx1, W, br_, gr_, ber_)

  sp = _segsum(ht, srcs, dsts, zeros)
  tx1, htt = _mid(sp, dis)
  sp = _segsum(htt, srcs, dsts, zeros)
  return _final(sp, dis, h, tx1, W4, b4r, Wr, brr)


# speedup vs baseline: 1.2592x; 1.2592x over previous
import functools

import jax
import jax.numpy as jnp
from jax import lax
from jax.experimental import pallas as pl
from jax.experimental.pallas import tpu as pltpu
from jax.experimental.pallas import tpu_sc as plsc

N = 10000
NP = 10112
H = 128
E = 640000
NSUB = 16
CHUNK = 64
KST = 8
QETOT = KST * CHUNK
NQ = 1280
EPAD = NQ * QETOT
ROWS_PER_SUB = NP // NSUB
NWQ = NQ // 32
NCH = NWQ * KST
QDEG = NWQ

_mesh = plsc.VectorSubcoreMesh(core_axis_name="c", subcore_axis_name="s")


@functools.partial(
    pl.kernel,
    out_type=jax.ShapeDtypeStruct((2, NP, H), jnp.float32),
    mesh=_mesh,
    scratch_types=[
        pltpu.VMEM((3, KST, CHUNK), jnp.int32),
        pltpu.VMEM((3, KST, CHUNK), jnp.int32),
        pltpu.VMEM((4, CHUNK, H), jnp.float32),
        pltpu.VMEM_SHARED((NP, H), jnp.float32),
        pltpu.SemaphoreType.DMA,
        pltpu.SemaphoreType.DMA,
        pltpu.SemaphoreType.DMA,
        pltpu.SemaphoreType.DMA,
        pltpu.SemaphoreType.DMA,
        pltpu.SemaphoreType.DMA,
        pltpu.SemaphoreType.DMA,
        pltpu.SemaphoreType.DMA,
        pltpu.SemaphoreType.DMA,
    ],
)
def _segsum(ht, srcs, dsts, zeros, out, sidx, didx, gbuf, acc,
            sg0, sg1, sg2, sg3, ss0, ss1, ss2, ss3, si):
  SG = (sg0, sg1, sg2, sg3)
  SS = (ss0, ss1, ss2, ss3)
  cid = lax.axis_index("c")
  sid = lax.axis_index("s")
  wid = sid * 2 + cid
  q0 = wid * NWQ
  r0 = sid * ROWS_PER_SUB
  pltpu.sync_copy(zeros.at[pl.ds(r0, ROWS_PER_SUB)],
                  acc.at[pl.ds(r0, ROWS_PER_SUB)])
  pltpu.sync_copy(srcs.at[q0], sidx.at[0])
  pltpu.sync_copy(dsts.at[q0], didx.at[0])
  pltpu.async_copy(srcs.at[q0 + 1], sidx.at[1], si)
  pltpu.async_copy(dsts.at[q0 + 1], didx.at[1], si)
  plsc.subcore_barrier()
  for c in range(3):
    pltpu.async_copy(ht.at[sidx.at[0, c]], gbuf.at[c], SG[c])

  def quantum(k, carry):
    slot = k % 3
    nslot = (k + 1) % 3

    @pl.when(k + 1 < NWQ)
    def _idx_arrive():
      pltpu.make_async_copy(srcs.at[q0], sidx.at[nslot], si).wait()
      pltpu.make_async_copy(dsts.at[q0], didx.at[nslot], si).wait()

    for r in range(KST):
      c = k * KST + r
      b = r % 4
      pltpu.make_async_copy(ht.at[sidx.at[slot, r]], gbuf.at[b],
                            SG[b]).wait()
      pltpu.async_copy(gbuf.at[b], acc.at[didx.at[slot, r]], SS[b],
                       add=True)

      @pl.when(c + 3 < NCH)
      def _issue():
        b3 = (r + 3) % 4

        @pl.when(c > 0)
        def _reuse():
          pltpu.make_async_copy(gbuf.at[b3], acc.at[didx.at[slot, r]],
                                SS[b3]).wait()

        if r < KST - 3:
          pltpu.async_copy(ht.at[sidx.at[slot, r + 3]], gbuf.at[b3],
                           SG[b3])
        else:
          pltpu.async_copy(ht.at[sidx.at[nslot, r + 3 - KST]], gbuf.at[b3],
                           SG[b3])

    @pl.when(k + 2 < NWQ)
    def _prefetch():
      pslot = (k + 2) % 3
      pltpu.async_copy(srcs.at[q0 + k + 2], sidx.at[pslot], si)
      pltpu.async_copy(dsts.at[q0 + k + 2], didx.at[pslot], si)

    return carry

  lax.fori_loop(0, NWQ, quantum, 0)
  for b in range(4):
    pltpu.make_async_copy(gbuf.at[b], acc.at[didx.at[0, 0]], SS[b]).wait()
  plsc.subcore_barrier()
  pltpu.sync_copy(acc.at[pl.ds(r0, ROWS_PER_SUB)],
                  out.at[cid, pl.ds(r0, ROWS_PER_SUB)])


@functools.partial(
    pl.kernel,
    out_type=jax.ShapeDtypeStruct((2, NP, H), jnp.float32),
    mesh=_mesh,
    scratch_types=[
        pltpu.VMEM((2, KST, CHUNK), jnp.int32),
        pltpu.VMEM((CHUNK, H), jnp.float32),
        pltpu.VMEM_SHARED((NP, H), jnp.float32),
        pltpu.SemaphoreType.DMA,
        pltpu.SemaphoreType.DMA,
    ],
)
def _deg_kernel(srcs, ones, zeros, out, sidx, obuf, acc, ss, si):
  cid = lax.axis_index("c")
  sid = lax.axis_index("s")
  q0 = (sid * 2 + cid) * QDEG
  r0 = sid * ROWS_PER_SUB
  pltpu.sync_copy(zeros.at[pl.ds(r0, ROWS_PER_SUB)],
                  acc.at[pl.ds(r0, ROWS_PER_SUB)])
  pltpu.sync_copy(ones, obuf)
  pltpu.sync_copy(srcs.at[q0], sidx.at[0])
  pltpu.async_copy(srcs.at[q0 + 1], sidx.at[1], si)
  plsc.subcore_barrier()

  def block(k, carry):
    slot = k % 2
    for r in range(KST):
      pltpu.async_copy(obuf, acc.at[sidx.at[slot, r]], ss, add=True)
    for r in range(KST):
      pltpu.make_async_copy(obuf, acc.at[sidx.at[slot, r]], ss).wait()

    @pl.when(k + 1 < QDEG)
    def _boundary():
      pltpu.make_async_copy(srcs.at[q0], sidx.at[1 - slot], si).wait()

      @pl.when(k + 2 < QDEG)
      def _prefetch():
        pltpu.async_copy(srcs.at[q0 + k + 2], sidx.at[slot], si)

    return carry

  lax.fori_loop(0, QDEG, block, 0)
  plsc.subcore_barrier()
  pltpu.sync_copy(acc.at[pl.ds(r0, ROWS_PER_SUB)],
                  out.at[cid, pl.ds(r0, ROWS_PER_SUB)])


_dot = functools.partial(lax.dot_general,
                         dimension_numbers=(((1,), (0,)), ((), ())),
                         preferred_element_type=jnp.float32)


def _prep_body(degp_ref, x16_ref, w1_ref, dis_ref, y2t_ref, y1_ref, y0c_ref):
  deg = degp_ref[0, :, 0:1] + degp_ref[1, :, 0:1]
  rowmask = lax.broadcasted_iota(jnp.int32, (NP, 1), 0) < N
  valid = jnp.logical_and(deg > 0.0, rowmask)
  dis = jnp.where(valid, lax.rsqrt(jnp.where(deg > 0.0, deg, 1.0)), 0.0)
  dis128 = jnp.broadcast_to(dis, (NP, H))
  dis_ref[...] = dis128
  x16 = x16_ref[...]
  y0 = _dot(x16, w1_ref[0])
  y1 = _dot(x16, w1_ref[1])
  y2 = _dot(x16, w1_ref[2])
  y2t_ref[...] = dis128 * y2
  y1_ref[...] = y1
  y0c_ref[...] = y0 - y2


_prep = pl.pallas_call(
    _prep_body,
    out_shape=[jax.ShapeDtypeStruct((NP, H), jnp.float32)] * 4,
)


def _mid1_body(sp_ref, dis_ref, y1_ref, zt_ref):
  d = dis_ref[...]
  zt_ref[...] = d * y1_ref[...] - 2.0 * d * d * (sp_ref[0] + sp_ref[1])


_mid1 = pl.pallas_call(
    _mid1_body,
    out_shape=jax.ShapeDtypeStruct((NP, H), jnp.float32),
)


def _bn_act(h, g, be, act):
  if act == "lrelu":
    h = jnp.where(h >= 0.0, h, 0.01 * h)
  else:
    h = jnp.maximum(h, 0.0)
  hv = h[0:N, :]
  mean = jnp.mean(hv, axis=0, keepdims=True)
  var = jnp.mean((hv - mean) ** 2, axis=0, keepdims=True)
  return (h - mean) * lax.rsqrt(var + 1e-5) * g + be


def _make_post1(act):
  def body(sp_ref, dis_ref, y0c_ref, b_ref, g_ref, be_ref, h_ref, ht_ref):
    dis = dis_ref[...]
    h = y0c_ref[...] - dis * (sp_ref[0] + sp_ref[1]) + b_ref[...]
    hbn = _bn_act(h, g_ref[...], be_ref[...], act)
    h_ref[...] = hbn
    ht_ref[...] = dis * hbn

  return pl.pallas_call(
      body,
      out_shape=[jax.ShapeDtypeStruct((NP, H), jnp.float32)] * 2,
  )


def _mid_body(sp_ref, dis_ref, tx1_ref, ht_ref):
  dis = dis_ref[...]
  tx1 = -dis * (sp_ref[0] + sp_ref[1])
  tx1_ref[...] = tx1
  ht_ref[...] = dis * tx1


_mid = pl.pallas_call(
    _mid_body,
    out_shape=[jax.ShapeDtypeStruct((NP, H), jnp.float32)] * 2,
)


def _cheb_out(sp_ref, dis_ref, tx0_ref, tx1_ref, w_ref, b_ref):
  tx0 = tx0_ref[...]
  tx2 = -2.0 * dis_ref[...] * (sp_ref[0] + sp_ref[1]) - tx0
  h = _dot(tx0, w_ref[0]) + _dot(tx1_ref[...], w_ref[1]) + _dot(tx2, w_ref[2])
  return h + b_ref[...]


def _make_post(act):
  def body(sp_ref, dis_ref, tx0_ref, tx1_ref, w_ref, b_ref, g_ref, be_ref,
           h_ref, ht_ref):
    h = _cheb_out(sp_ref, dis_ref, tx0_ref, tx1_ref, w_ref, b_ref)
    hbn = _bn_act(h, g_ref[...], be_ref[...], act)
    h_ref[...] = hbn
    ht_ref[...] = dis_ref[...] * hbn

  return pl.pallas_call(
      body,
      out_shape=[jax.ShapeDtypeStruct((NP, H), jnp.float32)] * 2,
  )


def _final_body(sp_ref, dis_ref, tx0_ref, tx1_ref, w_ref, b_ref, wr_ref,
                br_ref, out_ref):
  h = _cheb_out(sp_ref, dis_ref, tx0_ref, tx1_ref, w_ref, b_ref)
  h = h[0:N, :]
  nrm = jnp.sqrt(jnp.sum(h * h, axis=1, keepdims=True))
  h = h / jnp.maximum(nrm, 1e-12)
  out_ref[...] = _dot(h, wr_ref[...]) + br_ref[...]


_final = pl.pallas_call(
    _final_body,
    out_shape=jax.ShapeDtypeStruct((N, 3), jnp.float32),
)

_post1_lrelu = _make_post1("lrelu")
_post_lrelu = _make_post("lrelu")
_post_relu = _make_post("relu")


def kernel(x, edge_index, W1, b1, W2, b2, W3, b3, W4, b4,
           g1, be1, g2, be2, g3, be3, Wr, br):
  pad_e = EPAD - E
  src = jnp.concatenate([edge_index[0], jnp.full((pad_e,), N, jnp.int32)])
  dst = jnp.concatenate([edge_index[1], jnp.full((pad_e,), N, jnp.int32)])
  srcs = src.reshape(NQ, KST, CHUNK)
  dsts = dst.reshape(NQ, KST, CHUNK)

  x16 = jnp.pad(x, ((0, NP - N), (0, 13)))
  W1p = jnp.pad(W1, ((0, 0), (0, 13), (0, 0)))
  ones = jnp.ones((CHUNK, H), jnp.float32)
  zeros = jnp.zeros((NP, H), jnp.float32)
  b1r, b2r, b3r, b4r = (v.reshape(1, H) for v in (b1, b2, b3, b4))
  g1r, g2r, g3r = (v.reshape(1, H) for v in (g1, g2, g3))
  be1r, be2r, be3r = (v.reshape(1, H) for v in (be1, be2, be3))
  brr = br.reshape(1, 3)

  degp = _deg_kernel(srcs, ones, zeros)
  dis, y2t, y1, y0c = _prep(degp, x16, W1p)

  sp = _segsum(y2t, srcs, dsts, zeros)
  zt = _mid1(sp, dis, y1)
  sp = _segsum(zt, srcs, dsts, zeros)
  h, ht = _post1_lrelu(sp, dis, y0c, b1r, g1r, be1r)

  for (W, br_, gr_, ber_, post) in (
      (W2, b2r, g2r, be2r, _post_lrelu),
      (W3, b3r, g3r, be3r, _post_relu),
  ):
    sp = _segsum(ht, srcs, dsts, zeros)
    tx1, htt = _mid(sp, dis)
    sp = _segsum(htt, srcs, dsts, zeros)
    h, ht = post(sp, dis, h, tx1, W, br_, gr_, ber_)

  sp = _segsum(ht, srcs, dsts, zeros)
  tx1, htt = _mid(sp, dis)
  sp = _segsum(htt, srcs, dsts, zeros)
  return _final(sp, dis, h, tx1, W4, b4r, Wr, brr)
